# baseline (speedup 1.0000x reference)
import functools

import jax
import jax.numpy as jnp
from jax import lax
from jax.experimental import pallas as pl
from jax.experimental.pallas import tpu as pltpu
from jax.experimental.pallas import tpu_sc as plsc

N = 10000
E = 320000
D = 128
NPAD = 10240
NC, NS = 2, 16
NW = NC * NS
CH = 128
CPT = 4 * (-(-E // (NW * CH * 4)))
CPP = CPT // 2
EPT = CPT * CH
EPAD = NW * EPT
RPT = NPAD // NS
TRASH = N

_mesh = plsc.VectorSubcoreMesh(core_axis_name="c", subcore_axis_name="s")


def _cnt_body(dst_hbm, z128_hbm, ones_hbm, cnt_out_hbm, dst_v, ones_v,
              cnt_sh):
    c = lax.axis_index("c")
    s = lax.axis_index("s")
    wid = c * NS + s
    pltpu.sync_copy(dst_hbm.at[wid], dst_v)
    pltpu.sync_copy(ones_hbm, ones_v)
    pltpu.sync_copy(z128_hbm, cnt_sh.at[pl.ds(s * RPT, RPT)])
    plsc.subcore_barrier()

    def chunk(i, carry):
        pltpu.sync_copy(ones_v, cnt_sh.at[dst_v.at[i]], add=True)
        return carry

    lax.fori_loop(0, CPT, chunk, 0)
    plsc.subcore_barrier()
    pltpu.sync_copy(cnt_sh.at[pl.ds(s * RPT, RPT)],
                    cnt_out_hbm.at[c].at[pl.ds(s * RPT, RPT)])


def _agg_body(h_hbm, src_hbm, dst_hbm, z128_hbm, out_hbm, src_v, dst_v,
              rows0_v, rows1_v, acc_sh, sem0):
    c = lax.axis_index("c")
    s = lax.axis_index("s")
    wid = c * NS + s
    pltpu.sync_copy(z128_hbm, acc_sh.at[pl.ds(s * RPT, RPT)])
    plsc.subcore_barrier()

    for p in range(2):
        pltpu.sync_copy(src_hbm.at[wid].at[pl.ds(p * CPP, CPP)], src_v)
        pltpu.sync_copy(dst_hbm.at[wid].at[pl.ds(p * CPP, CPP)], dst_v)

        def chunk(j, carry):
            i = 2 * j
            pltpu.async_copy(h_hbm.at[src_v.at[i]], rows0_v, sem0)
            pltpu.async_copy(h_hbm.at[src_v.at[i + 1]], rows1_v, sem0)
            pltpu.make_async_copy(h_hbm.at[src_v.at[i]], rows0_v,
                                  sem0).wait()
            pltpu.make_async_copy(h_hbm.at[src_v.at[i + 1]], rows1_v,
                                  sem0).wait()
            pltpu.sync_copy(rows0_v, acc_sh.at[dst_v.at[i]], add=True)
            pltpu.sync_copy(rows1_v, acc_sh.at[dst_v.at[i + 1]], add=True)
            return carry

        lax.fori_loop(0, CPP // 2, chunk, 0)
    plsc.subcore_barrier()
    pltpu.sync_copy(acc_sh.at[pl.ds(s * RPT, RPT)],
                    out_hbm.at[c].at[pl.ds(s * RPT, RPT)])


_cnt_call = pl.kernel(
    _cnt_body, mesh=_mesh,
    out_type=jax.ShapeDtypeStruct((NC, NPAD, D), jnp.float32),
    scratch_types=[
        pltpu.VMEM((CPT, CH), jnp.int32),
        pltpu.VMEM((CH, D), jnp.float32),
        pltpu.VMEM_SHARED((NPAD, D), jnp.float32),
    ])

_agg_call = pl.kernel(
    _agg_body, mesh=_mesh,
    out_type=jax.ShapeDtypeStruct((NC, NPAD, D), jnp.float32),
    scratch_types=[
        pltpu.VMEM((CPP, CH), jnp.int32),
        pltpu.VMEM((CPP, CH), jnp.int32),
        pltpu.VMEM((CH, D), jnp.float32),
        pltpu.VMEM((CH, D), jnp.float32),
        pltpu.VMEM_SHARED((NPAD, D), jnp.float32),
        pltpu.SemaphoreType.DMA,
    ])


def _inv_body(c0, c1, o):
    cnt = c0[...] + c1[...]
    o[...] = (1.0 / jnp.maximum(cnt, 1.0))[:, :16]


def _inv_call(cnt0, cnt1):
    BN = 2048
    row = lambda i: (i, 0)
    return pl.pallas_call(
        _inv_body,
        grid=(NPAD // BN,),
        in_specs=[pl.BlockSpec((BN, D), row), pl.BlockSpec((BN, D), row)],
        out_specs=pl.BlockSpec((BN, 16), row),
        out_shape=jax.ShapeDtypeStruct((NPAD, 16), jnp.float32),
    )(cnt0, cnt1)


def _tc_body(final, p0, p1, inv16, h, wl, wr, b, fcw, fcb, o):
    inv = inv16[:, 0:1]
    agg = (p0[...] + p1[...]) * inv
    y = (jnp.dot(agg, wl[...], preferred_element_type=jnp.float32)
         + jnp.dot(h[...], wr[...], preferred_element_type=jnp.float32)
         + b[...])
    y = jnp.maximum(y, 0.0)
    if final:
        y = jnp.dot(y, fcw[...], preferred_element_type=jnp.float32) + fcb[...]
    o[...] = y


def _tc_layer(p0, p1, inv16, h, wlT, wrT, bl, fcw=None, fcb=None):
    final = fcw is not None
    BN = 1024
    grid = (NPAD // BN,)
    row = lambda i: (i, 0)
    fix = lambda i: (0, 0)
    if not final:
        fcw = jnp.zeros((1, D), jnp.float32)
        fcb = jnp.zeros((1, D), jnp.float32)
    in_specs = [
        pl.BlockSpec((BN, D), row), pl.BlockSpec((BN, D), row),
        pl.BlockSpec((BN, 16), row),
        pl.BlockSpec((BN, D), row),
        pl.BlockSpec((D, D), fix), pl.BlockSpec((D, D), fix),
        pl.BlockSpec((1, D), fix),
        pl.BlockSpec(fcw.shape, fix), pl.BlockSpec(fcb.shape, fix),
    ]
    return pl.pallas_call(
        functools.partial(_tc_body, final),
        grid=grid,
        in_specs=in_specs,
        out_specs=pl.BlockSpec((BN, D), row),
        out_shape=jax.ShapeDtypeStruct((NPAD, D), jnp.float32),
    )(p0, p1, inv16, h, wlT, wrT, bl, fcw, fcb)


def kernel(x, edge_index, Wl1, bl1, Wr1, Wl2, bl2, Wr2, Wl3, bl3, Wr3,
           Wl4, bl4, Wr4, Wl5, bl5, Wr5, Wl6, bl6, Wr6, fc_W, fc_b):
    src = edge_index[0].astype(jnp.int32)
    dst = edge_index[1].astype(jnp.int32)
    src3 = jnp.pad(src, (0, EPAD - E)).reshape(NW, CPT, CH)
    dst3 = jnp.pad(dst, (0, EPAD - E),
                   constant_values=TRASH).reshape(NW, CPT, CH)
    h = jnp.pad(x, ((0, NPAD - N), (0, 0)))
    z128 = jnp.zeros((RPT, D), jnp.float32)
    ones128 = jnp.ones((CH, D), jnp.float32)

    layers = [(Wl1, bl1, Wr1), (Wl2, bl2, Wr2), (Wl3, bl3, Wr3),
              (Wl4, bl4, Wr4), (Wl5, bl5, Wr5), (Wl6, bl6, Wr6)]
    fcw_pad = jnp.zeros((D, D), jnp.float32).at[:, :2].set(fc_W.T)
    fcb_pad = jnp.zeros((1, D), jnp.float32).at[0, :2].set(fc_b)

    cnts = _cnt_call(dst3, z128, ones128)
    inv16 = _inv_call(cnts[0], cnts[1])
    for li, (Wl, bl, Wr) in enumerate(layers):
        parts = _agg_call(h, src3, dst3, z128)
        kw = {}
        if li == 5:
            kw = dict(fcw=fcw_pad, fcb=fcb_pad)
        h = _tc_layer(parts[0], parts[1], inv16, h,
                      Wl.T, Wr.T, bl.reshape(1, D), **kw)
    return h[:N, :2]


# speedup vs baseline: 2.1819x; 2.1819x over previous
import functools

import jax
import jax.numpy as jnp
from jax import lax
from jax.experimental import pallas as pl
from jax.experimental.pallas import tpu as pltpu
from jax.experimental.pallas import tpu_sc as plsc

N = 10000
E = 320000
D = 128
NPAD = 10240
NC, NS = 2, 16
NW = NC * NS
CH = 128
CPT = 4 * (-(-E // (NW * CH * 4)))
CPP = CPT // 2
CPT0, CPT1 = 101, 56
assert NS * (CPT0 + CPT1) * CH >= E
ECAP = NS * (CPT0 + CPT1) * CH
EPT = CPT * CH
EPAD = NW * EPT
RPT = NPAD // NS
TRASH = N

_mesh = plsc.VectorSubcoreMesh(core_axis_name="c", subcore_axis_name="s")


def _cnt_body(dst_hbm, z128_hbm, ones_hbm, cnt_out_hbm, dst_v, ones_v,
              cnt_sh):
    c = lax.axis_index("c")
    s = lax.axis_index("s")
    wid = c * NS + s
    pltpu.sync_copy(dst_hbm.at[wid], dst_v)
    pltpu.sync_copy(ones_hbm, ones_v)
    pltpu.sync_copy(z128_hbm, cnt_sh.at[pl.ds(s * RPT, RPT)])
    plsc.subcore_barrier()

    def chunk(i, carry):
        pltpu.sync_copy(ones_v, cnt_sh.at[dst_v.at[i]], add=True)
        return carry

    lax.fori_loop(0, CPT, chunk, 0)
    plsc.subcore_barrier()
    pltpu.sync_copy(cnt_sh.at[pl.ds(s * RPT, RPT)],
                    cnt_out_hbm.at[c].at[pl.ds(s * RPT, RPT)])


def _agg_body(h_hbm, src0_hbm, dst0_hbm, src1_hbm, dst1_hbm, z128_hbm,
              out_hbm, src_v, dst_v, rows0_v, acc_sh, sem0):
    c = lax.axis_index("c")
    s = lax.axis_index("s")
    pltpu.sync_copy(z128_hbm, acc_sh.at[pl.ds(s * RPT, RPT)])

    def chunk(i, carry):
        pltpu.async_copy(h_hbm.at[src_v.at[i]], rows0_v, sem0).wait()
        pltpu.sync_copy(rows0_v, acc_sh.at[dst_v.at[i]], add=True)
        return carry

    @pl.when(c == 0)
    def _():
        pltpu.sync_copy(src0_hbm.at[s], src_v)
        pltpu.sync_copy(dst0_hbm.at[s], dst_v)
        plsc.subcore_barrier()
        lax.fori_loop(0, CPT0, chunk, 0)

    @pl.when(c == 1)
    def _():
        pltpu.sync_copy(src1_hbm.at[s], src_v.at[pl.ds(0, CPT1)])
        pltpu.sync_copy(dst1_hbm.at[s], dst_v.at[pl.ds(0, CPT1)])
        plsc.subcore_barrier()
        lax.fori_loop(0, CPT1, chunk, 0)

    plsc.subcore_barrier()
    pltpu.sync_copy(acc_sh.at[pl.ds(s * RPT, RPT)],
                    out_hbm.at[c].at[pl.ds(s * RPT, RPT)])


_cnt_call = pl.kernel(
    _cnt_body, mesh=_mesh,
    out_type=jax.ShapeDtypeStruct((NC, NPAD, D), jnp.float32),
    scratch_types=[
        pltpu.VMEM((CPT, CH), jnp.int32),
        pltpu.VMEM((CH, D), jnp.float32),
        pltpu.VMEM_SHARED((NPAD, D), jnp.float32),
    ])

_agg_call = pl.kernel(
    _agg_body, mesh=_mesh,
    out_type=jax.ShapeDtypeStruct((NC, NPAD, D), jnp.float32),
    scratch_types=[
        pltpu.VMEM((CPT0, CH), jnp.int32),
        pltpu.VMEM((CPT0, CH), jnp.int32),
        pltpu.VMEM((CH, D), jnp.float32),
        pltpu.VMEM_SHARED((NPAD, D), jnp.float32),
        pltpu.SemaphoreType.DMA,
    ])


def _inv_body(c0, c1, o):
    cnt = c0[...] + c1[...]
    o[...] = (1.0 / jnp.maximum(cnt, 1.0))[:, :16]


def _inv_call(cnt0, cnt1):
    BN = 2048
    row = lambda i: (i, 0)
    return pl.pallas_call(
        _inv_body,
        grid=(NPAD // BN,),
        in_specs=[pl.BlockSpec((BN, D), row), pl.BlockSpec((BN, D), row)],
        out_specs=pl.BlockSpec((BN, 16), row),
        out_shape=jax.ShapeDtypeStruct((NPAD, 16), jnp.float32),
    )(cnt0, cnt1)


def _tc_body(final, p0, p1, inv16, h, wl, wr, b, fcw, fcb, o):
    inv = inv16[:, 0:1]
    agg = (p0[...] + p1[...]) * inv
    y = (jnp.dot(agg, wl[...], preferred_element_type=jnp.float32)
         + jnp.dot(h[...], wr[...], preferred_element_type=jnp.float32)
         + b[...])
    y = jnp.maximum(y, 0.0)
    if final:
        y = jnp.dot(y, fcw[...], preferred_element_type=jnp.float32) + fcb[...]
    o[...] = y


def _tc_layer(p0, p1, inv16, h, wlT, wrT, bl, fcw=None, fcb=None):
    final = fcw is not None
    BN = 1024
    grid = (NPAD // BN,)
    row = lambda i: (i, 0)
    fix = lambda i: (0, 0)
    if not final:
        fcw = jnp.zeros((1, D), jnp.float32)
        fcb = jnp.zeros((1, D), jnp.float32)
    in_specs = [
        pl.BlockSpec((BN, D), row), pl.BlockSpec((BN, D), row),
        pl.BlockSpec((BN, 16), row),
        pl.BlockSpec((BN, D), row),
        pl.BlockSpec((D, D), fix), pl.BlockSpec((D, D), fix),
        pl.BlockSpec((1, D), fix),
        pl.BlockSpec(fcw.shape, fix), pl.BlockSpec(fcb.shape, fix),
    ]
    return pl.pallas_call(
        functools.partial(_tc_body, final),
        grid=grid,
        in_specs=in_specs,
        out_specs=pl.BlockSpec((BN, D), row),
        out_shape=jax.ShapeDtypeStruct((NPAD, D), jnp.float32),
    )(p0, p1, inv16, h, wlT, wrT, bl, fcw, fcb)


def kernel(x, edge_index, Wl1, bl1, Wr1, Wl2, bl2, Wr2, Wl3, bl3, Wr3,
           Wl4, bl4, Wr4, Wl5, bl5, Wr5, Wl6, bl6, Wr6, fc_W, fc_b):
    src = edge_index[0].astype(jnp.int32)
    dst = edge_index[1].astype(jnp.int32)
    dst3 = jnp.pad(dst, (0, EPAD - E),
                   constant_values=TRASH).reshape(NW, CPT, CH)
    srcf = jnp.pad(src, (0, ECAP - E))
    dstf = jnp.pad(dst, (0, ECAP - E), constant_values=TRASH)
    cut = NS * CPT0 * CH
    src0 = srcf[:cut].reshape(NS, CPT0, CH)
    dst0 = dstf[:cut].reshape(NS, CPT0, CH)
    src1 = srcf[cut:].reshape(NS, CPT1, CH)
    dst1 = dstf[cut:].reshape(NS, CPT1, CH)
    h = jnp.pad(x, ((0, NPAD - N), (0, 0)))
    z128 = jnp.zeros((RPT, D), jnp.float32)
    ones128 = jnp.ones((CH, D), jnp.float32)

    layers = [(Wl1, bl1, Wr1), (Wl2, bl2, Wr2), (Wl3, bl3, Wr3),
              (Wl4, bl4, Wr4), (Wl5, bl5, Wr5), (Wl6, bl6, Wr6)]
    fcw_pad = jnp.zeros((D, D), jnp.float32).at[:, :2].set(fc_W.T)
    fcb_pad = jnp.zeros((1, D), jnp.float32).at[0, :2].set(fc_b)

    cnts = _cnt_call(dst3, z128, ones128)
    inv16 = _inv_call(cnts[0], cnts[1])
    for li, (Wl, bl, Wr) in enumerate(layers):
        parts = _agg_call(h, src0, dst0, src1, dst1, z128)
        kw = {}
        if li == 5:
            kw = dict(fcw=fcw_pad, fcb=fcb_pad)
        h = _tc_layer(parts[0], parts[1], inv16, h,
                      Wl.T, Wr.T, bl.reshape(1, D), **kw)
    return h[:N, :2]


# speedup vs baseline: 2.2489x; 1.0307x over previous
import functools

import jax
import jax.numpy as jnp
from jax import lax
from jax.experimental import pallas as pl
from jax.experimental.pallas import tpu as pltpu
from jax.experimental.pallas import tpu_sc as plsc

N = 10000
E = 320000
D = 128
NPAD = 10240
NC, NS = 2, 16
NW = NC * NS
CH = 128
CPT = 4 * (-(-E // (NW * CH * 4)))
CPP = CPT // 2
CPT0, CPT1 = 95, 62
assert NS * (CPT0 + CPT1) * CH >= E
ECAP = NS * (CPT0 + CPT1) * CH
EPT = CPT * CH
EPAD = NW * EPT
RPT = NPAD // NS
TRASH = N

_mesh = plsc.VectorSubcoreMesh(core_axis_name="c", subcore_axis_name="s")


def _cnt_body(dst_hbm, z128_hbm, ones_hbm, cnt_out_hbm, dst_v, ones_v,
              cnt_sh):
    c = lax.axis_index("c")
    s = lax.axis_index("s")
    wid = c * NS + s
    pltpu.sync_copy(dst_hbm.at[wid], dst_v)
    pltpu.sync_copy(ones_hbm, ones_v)
    pltpu.sync_copy(z128_hbm, cnt_sh.at[pl.ds(s * RPT, RPT)])
    plsc.subcore_barrier()

    def chunk(i, carry):
        pltpu.sync_copy(ones_v, cnt_sh.at[dst_v.at[i]], add=True)
        return carry

    lax.fori_loop(0, CPT, chunk, 0)
    plsc.subcore_barrier()
    pltpu.sync_copy(cnt_sh.at[pl.ds(s * RPT, RPT)],
                    cnt_out_hbm.at[c].at[pl.ds(s * RPT, RPT)])


def _agg_body(h_hbm, src0_hbm, dst0_hbm, src1_hbm, dst1_hbm, z128_hbm,
              out_hbm, src_v, dst_v, rows0_v, acc_sh, sem0):
    c = lax.axis_index("c")
    s = lax.axis_index("s")
    pltpu.sync_copy(z128_hbm, acc_sh.at[pl.ds(s * RPT, RPT)])

    def chunk(i, carry):
        pltpu.async_copy(h_hbm.at[src_v.at[i]], rows0_v, sem0).wait()
        pltpu.sync_copy(rows0_v, acc_sh.at[dst_v.at[i]], add=True)
        return carry

    @pl.when(c == 0)
    def _():
        pltpu.sync_copy(src0_hbm.at[s], src_v)
        pltpu.sync_copy(dst0_hbm.at[s], dst_v)
        plsc.subcore_barrier()
        lax.fori_loop(0, CPT0, chunk, 0)

    @pl.when(c == 1)
    def _():
        pltpu.sync_copy(src1_hbm.at[s], src_v.at[pl.ds(0, CPT1)])
        pltpu.sync_copy(dst1_hbm.at[s], dst_v.at[pl.ds(0, CPT1)])
        plsc.subcore_barrier()
        lax.fori_loop(0, CPT1, chunk, 0)

    plsc.subcore_barrier()
    pltpu.sync_copy(acc_sh.at[pl.ds(s * RPT, RPT)],
                    out_hbm.at[c].at[pl.ds(s * RPT, RPT)])


_cnt_call = pl.kernel(
    _cnt_body, mesh=_mesh,
    out_type=jax.ShapeDtypeStruct((NC, NPAD, D), jnp.float32),
    scratch_types=[
        pltpu.VMEM((CPT, CH), jnp.int32),
        pltpu.VMEM((CH, D), jnp.float32),
        pltpu.VMEM_SHARED((NPAD, D), jnp.float32),
    ])

_agg_call = pl.kernel(
    _agg_body, mesh=_mesh,
    out_type=jax.ShapeDtypeStruct((NC, NPAD, D), jnp.float32),
    scratch_types=[
        pltpu.VMEM((CPT0, CH), jnp.int32),
        pltpu.VMEM((CPT0, CH), jnp.int32),
        pltpu.VMEM((CH, D), jnp.float32),
        pltpu.VMEM_SHARED((NPAD, D), jnp.float32),
        pltpu.SemaphoreType.DMA,
    ])


def _inv_body(c0, c1, o):
    cnt = c0[...] + c1[...]
    o[...] = (1.0 / jnp.maximum(cnt, 1.0))[:, :16]


def _inv_call(cnt0, cnt1):
    BN = 2048
    row = lambda i: (i, 0)
    return pl.pallas_call(
        _inv_body,
        grid=(NPAD // BN,),
        in_specs=[pl.BlockSpec((BN, D), row), pl.BlockSpec((BN, D), row)],
        out_specs=pl.BlockSpec((BN, 16), row),
        out_shape=jax.ShapeDtypeStruct((NPAD, 16), jnp.float32),
    )(cnt0, cnt1)


def _tc_body(final, p0, p1, inv16, h, wl, wr, b, fcw, fcb, o):
    inv = inv16[:, 0:1]
    agg = (p0[...] + p1[...]) * inv
    y = (jnp.dot(agg, wl[...], preferred_element_type=jnp.float32)
         + jnp.dot(h[...], wr[...], preferred_element_type=jnp.float32)
         + b[...])
    y = jnp.maximum(y, 0.0)
    if final:
        y = jnp.dot(y, fcw[...], preferred_element_type=jnp.float32) + fcb[...]
    o[...] = y


def _tc_layer(p0, p1, inv16, h, wlT, wrT, bl, fcw=None, fcb=None):
    final = fcw is not None
    BN = 1024
    grid = (NPAD // BN,)
    row = lambda i: (i, 0)
    fix = lambda i: (0, 0)
    if not final:
        fcw = jnp.zeros((1, D), jnp.float32)
        fcb = jnp.zeros((1, D), jnp.float32)
    in_specs = [
        pl.BlockSpec((BN, D), row), pl.BlockSpec((BN, D), row),
        pl.BlockSpec((BN, 16), row),
        pl.BlockSpec((BN, D), row),
        pl.BlockSpec((D, D), fix), pl.BlockSpec((D, D), fix),
        pl.BlockSpec((1, D), fix),
        pl.BlockSpec(fcw.shape, fix), pl.BlockSpec(fcb.shape, fix),
    ]
    return pl.pallas_call(
        functools.partial(_tc_body, final),
        grid=grid,
        in_specs=in_specs,
        out_specs=pl.BlockSpec((BN, D), row),
        out_shape=jax.ShapeDtypeStruct((NPAD, D), jnp.float32),
    )(p0, p1, inv16, h, wlT, wrT, bl, fcw, fcb)


def kernel(x, edge_index, Wl1, bl1, Wr1, Wl2, bl2, Wr2, Wl3, bl3, Wr3,
           Wl4, bl4, Wr4, Wl5, bl5, Wr5, Wl6, bl6, Wr6, fc_W, fc_b):
    src = edge_index[0].astype(jnp.int32)
    dst = edge_index[1].astype(jnp.int32)
    dst3 = jnp.pad(dst, (0, EPAD - E),
                   constant_values=TRASH).reshape(NW, CPT, CH)
    srcf = jnp.pad(src, (0, ECAP - E))
    dstf = jnp.pad(dst, (0, ECAP - E), constant_values=TRASH)
    cut = NS * CPT0 * CH
    src0 = srcf[:cut].reshape(NS, CPT0, CH)
    dst0 = dstf[:cut].reshape(NS, CPT0, CH)
    src1 = srcf[cut:].reshape(NS, CPT1, CH)
    dst1 = dstf[cut:].reshape(NS, CPT1, CH)
    h = jnp.pad(x, ((0, NPAD - N), (0, 0)))
    z128 = jnp.zeros((RPT, D), jnp.float32)
    ones128 = jnp.ones((CH, D), jnp.float32)

    layers = [(Wl1, bl1, Wr1), (Wl2, bl2, Wr2), (Wl3, bl3, Wr3),
              (Wl4, bl4, Wr4), (Wl5, bl5, Wr5), (Wl6, bl6, Wr6)]
    fcw_pad = jnp.zeros((D, D), jnp.float32).at[:, :2].set(fc_W.T)
    fcb_pad = jnp.zeros((1, D), jnp.float32).at[0, :2].set(fc_b)

    cnts = _cnt_call(dst3, z128, ones128)
    inv16 = _inv_call(cnts[0], cnts[1])
    for li, (Wl, bl, Wr) in enumerate(layers):
        parts = _agg_call(h, src0, dst0, src1, dst1, z128)
        kw = {}
        if li == 5:
            kw = dict(fcw=fcw_pad, fcb=fcb_pad)
        h = _tc_layer(parts[0], parts[1], inv16, h,
                      Wl.T, Wr.T, bl.reshape(1, D), **kw)
    return h[:N, :2]
